# baseline (speedup 1.0000x reference)
import jax
import jax.numpy as jnp
from jax import lax
from jax.experimental import pallas as pl

_ROWS = 3072


_RG = 16


def _onehot_argmax_block(x_ref, o_ref):
    n = x_ref.shape[1]
    iota = lax.broadcasted_iota(jnp.int32, (_RG, n), 1)

    def group(i, c):
        xs = x_ref[pl.ds(i * _RG, _RG), :]
        m = jnp.max(xs, axis=1, keepdims=True)
        idx = jnp.min(jnp.where(xs == m, iota, n), axis=1, keepdims=True)
        o_ref[pl.ds(i * _RG, _RG), :] = (iota == idx).astype(o_ref.dtype)
        return c

    lax.fori_loop(0, _ROWS // _RG, group, 0)


def kernel(x):
    b, s, n = x.shape
    rows = b * s
    x2 = x.reshape(rows, n)
    out = pl.pallas_call(
        _onehot_argmax_block,
        grid=(rows // _ROWS,),
        in_specs=[pl.BlockSpec((_ROWS, n), lambda i: (i, 0))],
        out_specs=pl.BlockSpec((_ROWS, n), lambda i: (i, 0)),
        out_shape=jax.ShapeDtypeStruct((rows, n), x.dtype),
    )(x2)
    return out.reshape(b, s, n)


# speedup vs baseline: 4.0985x; 4.0985x over previous
import jax
import jax.numpy as jnp
from jax import lax
from jax.experimental import pallas as pl
from jax.experimental.pallas import tpu as pltpu

_ROWS = 3072
_N = 1024
_CSPLIT = 2
_CW = _N // _CSPLIT


def _onehot_argmax_block(x_ref, o_ref, idx_ref):
    j = pl.program_id(1)

    @pl.when(j == 0)
    def _():
        x = x_ref[...]
        m = jnp.max(x, axis=1, keepdims=True)
        iota = lax.broadcasted_iota(jnp.int32, x.shape, 1)
        idx_ref[...] = jnp.min(jnp.where(x == m, iota, _N), axis=1, keepdims=True)

    iota_c = lax.broadcasted_iota(jnp.int32, (_ROWS, _CW), 1) + j * _CW
    o_ref[...] = (iota_c == idx_ref[...]).astype(o_ref.dtype)


def kernel(x):
    b, s, n = x.shape
    rows = b * s
    x2 = x.reshape(rows, n)
    out = pl.pallas_call(
        _onehot_argmax_block,
        grid=(rows // _ROWS, _CSPLIT),
        in_specs=[pl.BlockSpec((_ROWS, n), lambda i, j: (i, 0))],
        out_specs=pl.BlockSpec((_ROWS, _CW), lambda i, j: (i, j)),
        out_shape=jax.ShapeDtypeStruct((rows, n), x.dtype),
        scratch_shapes=[pltpu.VMEM((_ROWS, 1), jnp.int32)],
    )(x2)
    return out.reshape(b, s, n)


# speedup vs baseline: 6.1268x; 1.4949x over previous
import jax
import jax.numpy as jnp
from jax import lax
from jax.experimental import pallas as pl

_ROWS = 3072


def _onehot_argmax_block(x_ref, o_ref):
    x = x_ref[...]
    n = x.shape[1]
    m = jnp.max(x, axis=1, keepdims=True)
    iota = lax.broadcasted_iota(jnp.int32, x.shape, 1)
    idx = jnp.min(jnp.where(x == m, iota, n), axis=1, keepdims=True)
    o_ref[...] = (iota == idx).astype(o_ref.dtype)


def kernel(x):
    b, s, n = x.shape
    rows = b * s
    x2 = x.reshape(rows, n)
    out = pl.pallas_call(
        _onehot_argmax_block,
        grid=(rows // _ROWS,),
        in_specs=[pl.BlockSpec((_ROWS, n), lambda i: (i, 0))],
        out_specs=pl.BlockSpec((_ROWS, n), lambda i: (i, 0)),
        out_shape=jax.ShapeDtypeStruct((rows, n), x.dtype),
    )(x2)
    return out.reshape(b, s, n)
